# baseline (speedup 1.0000x reference)
import functools

import jax
import jax.numpy as jnp
from jax import lax
from jax.experimental import pallas as pl
from jax.experimental.pallas import tpu as pltpu
from jax.experimental.pallas import tpu_sc as plsc

_NUM_CLASSES = 3
_NUM_ANCHORS = 3
_G = 76
_S = _G * _G
_NCH = 7 + _NUM_CLASSES
_STRIDE = 8.0
_ANCHOR_W = (11.0, 23.0, 37.0)
_ANCHOR_H = (14.0, 27.0, 58.0)

_B = 64
_UNITS = _B * _NUM_ANCHORS
_NW = 32
_UPW = _UNITS // _NW
_SCHUNK = 304
_NCHUNK = _S // _SCHUNK
_GPC = _SCHUNK // 16


def _sigmoid(v):
    return 1.0 / (1.0 + jnp.exp(-v))


def _sc_body(x_hbm, out_hbm, in_v, out_v):
    cid = lax.axis_index("c")
    sid = lax.axis_index("s")
    wid = sid * 2 + cid
    iota16 = lax.iota(jnp.int32, 16)

    for k in range(_UPW):
        u = wid * _UPW + k
        pltpu.sync_copy(x_hbm.at[u], in_v)
        aw = _ANCHOR_W[k % _NUM_ANCHORS]
        ah = _ANCHOR_H[k % _NUM_ANCHORS]

        def chunk_body(ch, carry, aw=aw, ah=ah, u=u):
            def grp_body(g, carry2, aw=aw, ah=ah, ch=ch):
                base = ch * _SCHUNK + g * 16
                s_loc = g * 16 + iota16
                gx = (s_loc % _G).astype(jnp.float32)
                gy = (ch * 4 + s_loc // _G).astype(jnp.float32)

                v0 = in_v[0, pl.ds(base, 16)]
                r0 = (_sigmoid(v0) + gx) * _STRIDE
                plsc.store_scatter(out_v, [s_loc, jnp.full((16,), 0, jnp.int32)], r0)

                v1 = in_v[1, pl.ds(base, 16)]
                r1 = (_sigmoid(v1) + gy) * _STRIDE
                plsc.store_scatter(out_v, [s_loc, jnp.full((16,), 1, jnp.int32)], r1)

                v2 = in_v[2, pl.ds(base, 16)]
                r2 = jnp.minimum(jnp.exp(v2), 1000.0) * aw
                plsc.store_scatter(out_v, [s_loc, jnp.full((16,), 2, jnp.int32)], r2)

                v3 = in_v[3, pl.ds(base, 16)]
                r3 = jnp.minimum(jnp.exp(v3), 1000.0) * ah
                plsc.store_scatter(out_v, [s_loc, jnp.full((16,), 3, jnp.int32)], r3)

                for c in (4, 5):
                    vc = in_v[c, pl.ds(base, 16)]
                    plsc.store_scatter(
                        out_v, [s_loc, jnp.full((16,), c, jnp.int32)], vc
                    )

                for c in (6, 7, 8, 9):
                    vc = in_v[c, pl.ds(base, 16)]
                    plsc.store_scatter(
                        out_v, [s_loc, jnp.full((16,), c, jnp.int32)], _sigmoid(vc)
                    )
                return carry2

            lax.fori_loop(0, _GPC, grp_body, 0)
            pltpu.sync_copy(out_v, out_hbm.at[u, pl.ds(ch * _SCHUNK, _SCHUNK)])
            return carry

        lax.fori_loop(0, _NCHUNK, chunk_body, 0)


@jax.jit
def kernel(x):
    B = x.shape[0]
    xr = x.reshape(_UNITS, _NCH, _S)
    run = pl.kernel(
        _sc_body,
        mesh=plsc.VectorSubcoreMesh(core_axis_name="c", subcore_axis_name="s"),
        out_type=jax.ShapeDtypeStruct((_UNITS, _S, _NCH), jnp.float32),
        scratch_types=[
            pltpu.VMEM((_NCH, _S), jnp.float32),
            pltpu.VMEM((_SCHUNK, _NCH), jnp.float32),
        ],
        compiler_params=pltpu.CompilerParams(
            needs_layout_passes=False, use_tc_tiling_on_sc=False
        ),
    )
    out = run(xr)
    return out.reshape(B, _NUM_ANCHORS * _S, _NCH)
